# baseline (speedup 1.0000x reference)
import functools

import jax
import jax.numpy as jnp
from jax import lax
from jax.experimental import pallas as pl
from jax.experimental.pallas import tpu as pltpu
from jax.experimental.pallas import tpu_sc as plsc

BATCH = 16384
D = 64
NC = 2
NS = 16
NW = NC * NS
BPW = BATCH // NW
C = 128
NCH = BPW // C

_mesh = plsc.VectorSubcoreMesh(core_axis_name="c", subcore_axis_name="s")


@functools.partial(
    pl.kernel,
    mesh=_mesh,
    compiler_params=pltpu.CompilerParams(use_tc_tiling_on_sc=False),
    out_type=jax.ShapeDtypeStruct((BATCH, 16), jnp.float32),
    scratch_types=[
        pltpu.VMEM((NCH, C), jnp.int32),
        pltpu.VMEM((NCH, C), jnp.int32),
        pltpu.VMEM((NCH, C), jnp.int32),
        pltpu.VMEM((C, D), jnp.float32),
        pltpu.VMEM((C, D), jnp.float32),
        pltpu.VMEM((C, D), jnp.float32),
        pltpu.VMEM((C, D), jnp.float32),
        pltpu.VMEM((C, D), jnp.float32),
        pltpu.VMEM((C, D), jnp.float32),
        pltpu.VMEM((BPW, 16), jnp.float32),
        pltpu.SemaphoreType.DMA,
    ],
)
def _complex_partial_kernel(h_hbm, r_hbm, t_hbm, ere_hbm, eim_hbm, rre_hbm,
                            rim_hbm, out_hbm, hi_v, ri_v, ti_v, hre_v, him_v,
                            tre_v, tim_v, rre_v, rim_v, pacc_v, sem):
    cid = lax.axis_index("c")
    sid = lax.axis_index("s")
    wid = sid * NC + cid

    pltpu.sync_copy(h_hbm.at[wid], hi_v)
    pltpu.sync_copy(r_hbm.at[wid], ri_v)
    pltpu.sync_copy(t_hbm.at[wid], ti_v)

    for ch in range(NCH):
        cp1 = pltpu.async_copy(ere_hbm.at[hi_v.at[ch]], hre_v, sem)
        cp2 = pltpu.async_copy(eim_hbm.at[hi_v.at[ch]], him_v, sem)
        cp3 = pltpu.async_copy(ere_hbm.at[ti_v.at[ch]], tre_v, sem)
        cp4 = pltpu.async_copy(eim_hbm.at[ti_v.at[ch]], tim_v, sem)
        cp5 = pltpu.async_copy(rre_hbm.at[ri_v.at[ch]], rre_v, sem)
        cp6 = pltpu.async_copy(rim_hbm.at[ri_v.at[ch]], rim_v, sem)
        cp1.wait()
        cp2.wait()
        cp3.wait()
        cp4.wait()
        cp5.wait()
        cp6.wait()

        def row_body(row, carry):
            acc = jnp.zeros((16,), jnp.float32)
            for j in range(D // 16):
                sl = pl.ds(j * 16, 16)
                a = hre_v[row, sl]
                b = him_v[row, sl]
                c = tre_v[row, sl]
                d = tim_v[row, sl]
                p = rre_v[row, sl]
                q = rim_v[row, sl]
                acc = acc + p * (a * c + b * d) + q * (a * d - b * c)
            pacc_v[ch * C + row, :] = acc
            return carry

        lax.fori_loop(0, C, row_body, 0)

    pltpu.sync_copy(pacc_v, out_hbm.at[pl.ds(wid * BPW, BPW)])


def _reduce_body(x_ref, o_ref):
    x = x_ref[...]
    o_ref[...] = -jnp.sum(x.reshape(BATCH // 8, 8, 16), axis=-1)


_reduce_call = pl.pallas_call(
    _reduce_body,
    out_shape=jax.ShapeDtypeStruct((BATCH // 8, 8), jnp.float32),
)


def kernel(h, r, t, entity_re, entity_im, relation_re, relation_im):
    h3 = h.astype(jnp.int32).reshape(NW, NCH, C)
    r3 = r.astype(jnp.int32).reshape(NW, NCH, C)
    t3 = t.astype(jnp.int32).reshape(NW, NCH, C)
    partial = _complex_partial_kernel(h3, r3, t3, entity_re, entity_im,
                                      relation_re, relation_im)
    out = _reduce_call(partial.reshape(BATCH // 8, 128))
    return out.reshape(BATCH)
